# baseline (speedup 1.0000x reference)
import jax
import jax.numpy as jnp
from jax.experimental import pallas as pl
from jax.experimental.pallas import tpu as pltpu

_FEAT = 1690
_BB = 128


def _banded(w, h_in):
    O, C, K, _ = w.shape
    h_out = h_in - K + 1
    i = jnp.arange(h_out)
    r = jnp.arange(h_in)
    ki = r[None, :] - i[:, None]
    valid = (ki >= 0) & (ki < K)
    a = w[:, :, jnp.clip(ki, 0, K - 1), :]
    a = jnp.where(valid[None, None, :, :, None], a, 0.0)
    a = a.transpose(0, 2, 4, 1, 3)
    return a.reshape(O * h_out, K * C * h_in)


def _rotl(x, k):
    return jnp.concatenate([x[:, k:], x[:, :k]], axis=1)


def _fwd_kernel(x_ref, ds_ref, a1e_ref, a1o_ref, b1r_ref,
                a2e_ref, a2o_ref, b2r_ref,
                w1t_ref, w1bt_ref, b1t_ref, w2t_ref, b2t_ref,
                w3t_ref, b3t_ref, w4t_ref, b4t_ref, w6t_ref, b6t_ref,
                o_ref):
    def mm(a, b):
        return jnp.dot(a, b, preferred_element_type=jnp.float32)

    x = jnp.transpose(x_ref[0], (1, 0)).reshape(64, 64 * _BB)

    xs1 = jnp.concatenate(
        [x] + [_rotl(x, kj * 128) for kj in range(1, 5)], axis=0)[:, :60 * _BB]
    b1r = b1r_ref[...]
    m1 = jnp.maximum(jnp.maximum(mm(a1e_ref[...], xs1), mm(a1o_ref[...], xs1))
                     + b1r, 0.0).astype(jnp.bfloat16)

    q1 = jnp.maximum(m1, _rotl(m1, 128))

    q1p = jnp.concatenate(
        [q1, jnp.zeros((2, q1.shape[1]), jnp.bfloat16)], axis=0)
    xs2 = jnp.concatenate(
        [q1p[:, :53 * _BB]] +
        [_rotl(q1p, kj * 256)[:, :53 * _BB] for kj in range(1, 5)], axis=0)
    b2r = b2r_ref[...]
    m2 = jnp.maximum(jnp.maximum(mm(a2e_ref[...], xs2), mm(a2o_ref[...], xs2))
                     + b2r, 0.0).astype(jnp.bfloat16)

    q2 = jnp.maximum(m2, _rotl(m2, 256))

    q2p = jnp.concatenate(
        [q2, jnp.zeros((6, q2.shape[1]), jnp.bfloat16)], axis=0)
    feat = jnp.concatenate(
        [q2p[:, 512 * j3:512 * j3 + _BB] for j3 in range(13)], axis=0)

    ds = jnp.transpose(ds_ref[0], (1, 0))
    h = mm(w1t_ref[...], feat) + mm(w1bt_ref[...], ds) + b1t_ref[...]
    h = jnp.maximum(h, 0.0)
    h = jnp.maximum(mm(w2t_ref[...], h) + b2t_ref[...], 0.0)
    h = jnp.maximum(mm(w3t_ref[...], h) + b3t_ref[...], 0.0)
    h = jnp.maximum(mm(w4t_ref[...], h) + b4t_ref[...], 0.0)
    logits = mm(w6t_ref[...], h) + b6t_ref[...]

    mx = jnp.max(logits, axis=0, keepdims=True)
    e = jnp.exp(logits - mx)
    den = jnp.sum(e, axis=0, keepdims=True)
    sm = e * pl.reciprocal(den, approx=True)
    o_ref[...] = jnp.transpose(sm, (1, 0))


def kernel(state, conv1_w, conv1_b, conv2_w, conv2_b, w1, b1, w2, b2,
           w3, b3, w4, b4, w6, b6):
    B = state.shape[0]
    G = (B + _BB - 1) // _BB
    Bp = G * _BB
    N = w6.shape[1]

    st = state if Bp == B else jnp.pad(state, ((0, Bp - B), (0, 0)))
    ximg = st[:, 3:].astype(jnp.bfloat16).reshape(G, _BB, 4096)
    dst = st[:, :3].reshape(G, _BB, 3)

    a1 = _banded(conv1_w, 64).astype(jnp.bfloat16)
    a1e, a1o = a1[0::2], a1[1::2]
    a2 = _banded(conv2_w, 30)
    a2s = jnp.pad(a2.reshape(260, 5, 150),
                  ((0, 0), (0, 0), (0, 2))).reshape(260, 760).astype(jnp.bfloat16)
    a2e, a2o = a2s[0::2], a2s[1::2]
    b1r = jnp.repeat(conv1_b, 30).reshape(150, 1)
    b2r = jnp.repeat(conv2_b, 13).reshape(130, 1)

    w1a = w1[:_FEAT].reshape(10, 13, 13, 512)
    w1a = w1a.transpose(2, 0, 1, 3).reshape(13, 130, 512)
    w1t = jnp.pad(w1a, ((0, 0), (0, 6), (0, 0))).reshape(13 * 136, 512)
    w1t = w1t.T.astype(jnp.bfloat16)
    w1bt = w1[_FEAT:].T

    c = lambda arr: pl.BlockSpec(arr.shape, lambda g: (0,) * arr.ndim)
    consts = (a1e, a1o, b1r, a2e, a2o, b2r,
              w1t, w1bt, b1.reshape(-1, 1), w2.T, b2.reshape(-1, 1),
              w3.T, b3.reshape(-1, 1), w4.T, b4.reshape(-1, 1),
              w6.T, b6.reshape(-1, 1))

    out = pl.pallas_call(
        _fwd_kernel,
        out_shape=jax.ShapeDtypeStruct((Bp, N), jnp.float32),
        grid=(G,),
        in_specs=[
            pl.BlockSpec((1, _BB, 4096), lambda g: (g, 0, 0)),
            pl.BlockSpec((1, _BB, 3), lambda g: (g, 0, 0)),
        ] + [c(a) for a in consts],
        out_specs=pl.BlockSpec((_BB, N), lambda g: (g, 0)),
        compiler_params=pltpu.CompilerParams(
            dimension_semantics=("parallel",)),
    )(ximg, dst, *consts)

    return out[:B]


# speedup vs baseline: 1.0163x; 1.0163x over previous
import jax
import jax.numpy as jnp
from jax.experimental import pallas as pl
from jax.experimental.pallas import tpu as pltpu

_FEAT = 1690
_BB = 128


def _banded(w, h_in):
    O, C, K, _ = w.shape
    h_out = h_in - K + 1
    i = jnp.arange(h_out)
    r = jnp.arange(h_in)
    ki = r[None, :] - i[:, None]
    valid = (ki >= 0) & (ki < K)
    a = w[:, :, jnp.clip(ki, 0, K - 1), :]
    a = jnp.where(valid[None, None, :, :, None], a, 0.0)
    a = a.transpose(0, 2, 4, 1, 3)
    return a.reshape(O * h_out, K * C * h_in)


def _rotl(x, k):
    return jnp.concatenate([x[:, k:], x[:, :k]], axis=1)


def _fwd_kernel(x_ref, ds_ref, a1e_ref, a1o_ref, b1r_ref,
                a2e_ref, a2o_ref, b2r_ref,
                w1t_ref, w1bt_ref, b1t_ref, w2t_ref, b2t_ref,
                w3t_ref, b3t_ref, w4t_ref, b4t_ref, w6t_ref, b6t_ref,
                o_ref):
    def mm(a, b):
        return jnp.dot(a, b, preferred_element_type=jnp.float32)

    x = jnp.transpose(x_ref[0], (1, 0)).reshape(64, 64 * _BB)

    xs1 = jnp.concatenate(
        [x] + [_rotl(x, kj * 128) for kj in range(1, 5)], axis=0)[:, :60 * _BB]
    b1r = b1r_ref[...]
    m1 = jnp.maximum(jnp.maximum(mm(a1e_ref[...], xs1), mm(a1o_ref[...], xs1))
                     + b1r, 0.0).astype(jnp.bfloat16)

    q1 = jnp.maximum(m1, _rotl(m1, 128))

    q1p = jnp.concatenate(
        [q1, jnp.zeros((2, q1.shape[1]), jnp.bfloat16)], axis=0)
    xs2 = jnp.concatenate(
        [q1p[:, :53 * _BB]] +
        [_rotl(q1p, kj * 256)[:, :53 * _BB] for kj in range(1, 5)], axis=0)
    b2r = b2r_ref[...]
    m2 = jnp.maximum(jnp.maximum(mm(a2e_ref[...], xs2), mm(a2o_ref[...], xs2))
                     + b2r, 0.0).astype(jnp.bfloat16)

    q2 = jnp.maximum(m2, _rotl(m2, 256))

    q2p = jnp.concatenate(
        [q2, jnp.zeros((6, q2.shape[1]), jnp.bfloat16)], axis=0)
    feat = jnp.concatenate(
        [q2p[:, 512 * j3:512 * j3 + _BB] for j3 in range(13)], axis=0)

    ds = jnp.transpose(ds_ref[0], (1, 0))
    h = mm(w1t_ref[...], feat) + mm(w1bt_ref[...], ds) + b1t_ref[...]
    h = jnp.maximum(h, 0.0)
    h = jnp.maximum(mm(w2t_ref[...], h) + b2t_ref[...], 0.0)
    h = jnp.maximum(mm(w3t_ref[...], h) + b3t_ref[...], 0.0)
    h = jnp.maximum(mm(w4t_ref[...], h) + b4t_ref[...], 0.0)
    logits = mm(w6t_ref[...], h) + b6t_ref[...]

    mx = jnp.max(logits, axis=0, keepdims=True)
    e = jnp.exp(logits - mx)
    den = jnp.sum(e, axis=0, keepdims=True)
    o_ref[0] = e * pl.reciprocal(den, approx=True)


def kernel(state, conv1_w, conv1_b, conv2_w, conv2_b, w1, b1, w2, b2,
           w3, b3, w4, b4, w6, b6):
    B = state.shape[0]
    G = (B + _BB - 1) // _BB
    Bp = G * _BB
    N = w6.shape[1]

    st = state if Bp == B else jnp.pad(state, ((0, Bp - B), (0, 0)))
    ximg = st[:, 3:].astype(jnp.bfloat16).reshape(G, _BB, 4096)
    dst = st[:, :3].reshape(G, _BB, 3)

    a1 = _banded(conv1_w, 64).astype(jnp.bfloat16)
    a1e, a1o = a1[0::2], a1[1::2]
    a2 = _banded(conv2_w, 30)
    a2s = jnp.pad(a2.reshape(260, 5, 150),
                  ((0, 0), (0, 0), (0, 2))).reshape(260, 760).astype(jnp.bfloat16)
    a2e, a2o = a2s[0::2], a2s[1::2]
    b1r = jnp.repeat(conv1_b, 30).reshape(150, 1)
    b2r = jnp.repeat(conv2_b, 13).reshape(130, 1)

    w1a = w1[:_FEAT].reshape(10, 13, 13, 512)
    w1a = w1a.transpose(2, 0, 1, 3).reshape(13, 130, 512)
    w1t = jnp.pad(w1a, ((0, 0), (0, 6), (0, 0))).reshape(13 * 136, 512)
    w1t = w1t.T.astype(jnp.bfloat16)
    w1bt = w1[_FEAT:].T

    c = lambda arr: pl.BlockSpec(arr.shape, lambda g: (0,) * arr.ndim)
    consts = (a1e, a1o, b1r, a2e, a2o, b2r,
              w1t, w1bt, b1.reshape(-1, 1), w2.T, b2.reshape(-1, 1),
              w3.T, b3.reshape(-1, 1), w4.T, b4.reshape(-1, 1),
              w6.T, b6.reshape(-1, 1))

    out = pl.pallas_call(
        _fwd_kernel,
        out_shape=jax.ShapeDtypeStruct((G, N, _BB), jnp.float32),
        grid=(G,),
        in_specs=[
            pl.BlockSpec((1, _BB, 4096), lambda g: (g, 0, 0)),
            pl.BlockSpec((1, _BB, 3), lambda g: (g, 0, 0)),
        ] + [c(a) for a in consts],
        out_specs=pl.BlockSpec((1, N, _BB), lambda g: (g, 0, 0)),
        compiler_params=pltpu.CompilerParams(
            dimension_semantics=("parallel",)),
    )(ximg, dst, *consts)

    return out.transpose(0, 2, 1).reshape(Bp, N)[:B]


# speedup vs baseline: 1.0322x; 1.0156x over previous
import jax
import jax.numpy as jnp
from jax.experimental import pallas as pl
from jax.experimental.pallas import tpu as pltpu

_FEAT = 1690
_BB = 128


def _banded(w, h_in):
    O, C, K, _ = w.shape
    h_out = h_in - K + 1
    i = jnp.arange(h_out)
    r = jnp.arange(h_in)
    ki = r[None, :] - i[:, None]
    valid = (ki >= 0) & (ki < K)
    a = w[:, :, jnp.clip(ki, 0, K - 1), :]
    a = jnp.where(valid[None, None, :, :, None], a, 0.0)
    a = a.transpose(0, 2, 4, 1, 3)
    return a.reshape(O * h_out, K * C * h_in)


def _rotl(x, k):
    return jnp.concatenate([x[:, k:], x[:, :k]], axis=1)


def _block_fwd(sub, x_ref, ds_ref, a1e_ref, a1o_ref, b1r_ref,
               a2e_ref, a2o_ref, b2r_ref,
               w1t_ref, w1bt_ref, b1t_ref, w2t_ref, b2t_ref,
               w3t_ref, b3t_ref, w4t_ref, b4t_ref, w6t_ref, b6t_ref,
               o_ref):
    def mm(a, b):
        return jnp.dot(a, b, preferred_element_type=jnp.float32)

    x = jnp.transpose(x_ref[sub], (1, 0)).reshape(64, 64 * _BB)

    xs1 = jnp.concatenate(
        [x] + [_rotl(x, kj * 128) for kj in range(1, 5)], axis=0)[:, :60 * _BB]
    b1r = b1r_ref[...]
    m1 = jnp.maximum(jnp.maximum(mm(a1e_ref[...], xs1), mm(a1o_ref[...], xs1))
                     + b1r, 0.0).astype(jnp.bfloat16)

    q1 = jnp.maximum(m1, _rotl(m1, 128))

    q1p = jnp.concatenate(
        [q1, jnp.zeros((2, q1.shape[1]), jnp.bfloat16)], axis=0)
    xs2 = jnp.concatenate(
        [q1p[:, :53 * _BB]] +
        [_rotl(q1p, kj * 256)[:, :53 * _BB] for kj in range(1, 5)], axis=0)
    b2r = b2r_ref[...]
    m2 = jnp.maximum(jnp.maximum(mm(a2e_ref[...], xs2), mm(a2o_ref[...], xs2))
                     + b2r, 0.0).astype(jnp.bfloat16)

    q2 = jnp.maximum(m2, _rotl(m2, 256))

    q2p = jnp.concatenate(
        [q2, jnp.zeros((6, q2.shape[1]), jnp.bfloat16)], axis=0)
    feat = jnp.concatenate(
        [q2p[:, 512 * j3:512 * j3 + _BB] for j3 in range(13)], axis=0)

    ds = jnp.transpose(ds_ref[sub], (1, 0))
    h = mm(w1t_ref[...], feat) + mm(w1bt_ref[...], ds) + b1t_ref[...]
    h = jnp.maximum(h, 0.0)
    h = jnp.maximum(mm(w2t_ref[...], h) + b2t_ref[...], 0.0)
    h = jnp.maximum(mm(w3t_ref[...], h) + b3t_ref[...], 0.0)
    h = jnp.maximum(mm(w4t_ref[...], h) + b4t_ref[...], 0.0)
    logits = mm(w6t_ref[...], h) + b6t_ref[...]

    mx = jnp.max(logits, axis=0, keepdims=True)
    e = jnp.exp(logits - mx)
    den = jnp.sum(e, axis=0, keepdims=True)
    o_ref[sub] = e * pl.reciprocal(den, approx=True)


def _fwd_kernel(*refs):
    _block_fwd(0, *refs)
    _block_fwd(1, *refs)


def kernel(state, conv1_w, conv1_b, conv2_w, conv2_b, w1, b1, w2, b2,
           w3, b3, w4, b4, w6, b6):
    B = state.shape[0]
    G = 2 * ((B + 2 * _BB - 1) // (2 * _BB))
    Bp = G * _BB
    N = w6.shape[1]

    st = state if Bp == B else jnp.pad(state, ((0, Bp - B), (0, 0)))
    ximg = st[:, 3:].astype(jnp.bfloat16).reshape(G, _BB, 4096)
    dst = st[:, :3].reshape(G, _BB, 3)

    a1 = _banded(conv1_w, 64).astype(jnp.bfloat16)
    a1e, a1o = a1[0::2], a1[1::2]
    a2 = _banded(conv2_w, 30)
    a2s = jnp.pad(a2.reshape(260, 5, 150),
                  ((0, 0), (0, 0), (0, 2))).reshape(260, 760).astype(jnp.bfloat16)
    a2e, a2o = a2s[0::2], a2s[1::2]
    b1r = jnp.repeat(conv1_b, 30).reshape(150, 1)
    b2r = jnp.repeat(conv2_b, 13).reshape(130, 1)

    w1a = w1[:_FEAT].reshape(10, 13, 13, 512)
    w1a = w1a.transpose(2, 0, 1, 3).reshape(13, 130, 512)
    w1t = jnp.pad(w1a, ((0, 0), (0, 6), (0, 0))).reshape(13 * 136, 512)
    w1t = w1t.T.astype(jnp.bfloat16)
    w1bt = w1[_FEAT:].T

    c = lambda arr: pl.BlockSpec(arr.shape, lambda g: (0,) * arr.ndim)
    consts = (a1e, a1o, b1r, a2e, a2o, b2r,
              w1t, w1bt, b1.reshape(-1, 1), w2.T, b2.reshape(-1, 1),
              w3.T, b3.reshape(-1, 1), w4.T, b4.reshape(-1, 1),
              w6.T, b6.reshape(-1, 1))

    out = pl.pallas_call(
        _fwd_kernel,
        out_shape=jax.ShapeDtypeStruct((G, N, _BB), jnp.float32),
        grid=(G // 2,),
        in_specs=[
            pl.BlockSpec((2, _BB, 4096), lambda g: (g, 0, 0)),
            pl.BlockSpec((2, _BB, 3), lambda g: (g, 0, 0)),
        ] + [c(a) for a in consts],
        out_specs=pl.BlockSpec((2, N, _BB), lambda g: (g, 0, 0)),
        compiler_params=pltpu.CompilerParams(
            dimension_semantics=("parallel",)),
    )(ximg, dst, *consts)

    return out.transpose(0, 2, 1).reshape(Bp, N)[:B]
